# baseline (speedup 1.0000x reference)
import functools
import jax
import jax.numpy as jnp
from jax import lax
from jax.experimental import pallas as pl
from jax.experimental.pallas import tpu as pltpu
from jax.experimental.pallas import tpu_sc as plsc

BATCH = 16384
FIELDS = 26
DIM = 64
NC = 2
NS = 16
NW = NC * NS
BLK = BATCH // NW
BW = 256
NCHUNK = BLK // BW
NTASK = FIELDS * NCHUNK


def _transpose_chunk(idx_v, f, c, g, t):
  iota = lax.iota(jnp.int32, 16)

  @pl.loop(0, BW // 16)
  def _(k):
    ids16 = plsc.load_gather(
        idx_v, [jnp.full((16,), f, jnp.int32), iota + (c * BW + k * 16)])
    basecol = (ids16 & 1) << 6
    rowv = iota + k * 16
    for d in range(DIM):
      t[d, pl.ds(k * 16, 16)] = plsc.load_gather(g, [rowv, basecol + d])


def _make_emb():
  mesh = plsc.VectorSubcoreMesh(core_axis_name="c", subcore_axis_name="s")

  @functools.partial(
      pl.kernel,
      mesh=mesh,
      out_type=jax.ShapeDtypeStruct((FIELDS, DIM, BATCH), jnp.float32),
      scratch_types=[
          pltpu.VMEM((FIELDS, BLK), jnp.int32),
          pltpu.VMEM((FIELDS * BLK,), jnp.int32),
          [pltpu.VMEM((BW, 128), jnp.float32) for _ in range(2)],
          [pltpu.VMEM((DIM, BW), jnp.float32) for _ in range(2)],
          [pltpu.SemaphoreType.DMA for _ in range(2)],
          [pltpu.SemaphoreType.DMA for _ in range(2)],
      ],
      compiler_params=pltpu.CompilerParams(needs_layout_passes=False),
  )
  def body(ids_hbm, jdx_hbm, w2_hbm, out_hbm, idx_v, jdx_v, gb, tb, gsems,
           wsems):
    wid = lax.axis_index("s") * NC + lax.axis_index("c")
    b0 = wid * BLK
    pltpu.sync_copy(ids_hbm.at[:, pl.ds(b0, BLK)], idx_v)
    for f in range(FIELDS):
      pltpu.async_copy(
          jdx_hbm.at[f, pl.ds(b0, BLK)], jdx_v.at[pl.ds(f * BLK, BLK)],
          wsems[0])
    for f in range(FIELDS):
      pltpu.make_async_copy(
          jdx_hbm.at[0, pl.ds(0, BLK)], jdx_v.at[pl.ds(0, BLK)],
          wsems[0]).wait()
    for p in range(2):
      pltpu.async_copy(
          w2_hbm.at[jdx_v.at[pl.ds(p * BW, BW)]], gb[p], gsems[p])

    @pl.loop(0, NTASK, step=2)
    def _(t0):
      f = t0 // 2
      for p in range(2):
        pltpu.make_async_copy(w2_hbm.at[pl.ds(0, BW)], gb[p], gsems[p]).wait()

        @pl.when(t0 + p >= 2)
        def _():
          pltpu.make_async_copy(
              tb[p], out_hbm.at[0, :, pl.ds(0, BW)], wsems[p]).wait()

        _transpose_chunk(idx_v, f, p, gb[p], tb[p])

        @pl.when(f + 1 < FIELDS)
        def _():
          pltpu.async_copy(
              w2_hbm.at[jdx_v.at[pl.ds((f + 1) * BLK + p * BW, BW)]],
              gb[p], gsems[p])

        pltpu.async_copy(
            tb[p], out_hbm.at[f, :, pl.ds(b0 + p * BW, BW)], wsems[p])

    for p in range(2):
      pltpu.make_async_copy(
          tb[p], out_hbm.at[0, :, pl.ds(0, BW)], wsems[p]).wait()

  return body


def kernel(input_ids, weight):
  ids_t = input_ids.T.astype(jnp.int32)
  jdx = lax.shift_right_logical(ids_t, 1)
  w2 = weight.reshape(-1, 128)
  out = _make_emb()(ids_t, jdx, w2)
  return out.transpose(2, 0, 1)


# speedup vs baseline: 1.2430x; 1.2430x over previous
import functools
import jax
import jax.numpy as jnp
from jax import lax
from jax.experimental import pallas as pl
from jax.experimental.pallas import tpu as pltpu
from jax.experimental.pallas import tpu_sc as plsc

BATCH = 16384
FIELDS = 26
DIM = 64
NC = 2
NS = 16
NW = NC * NS
BLK = BATCH // NW
BW = 256
NCHUNK = BLK // BW
NTASK = FIELDS * NCHUNK


def _transpose_chunk(idx_v, f, c, g, t):
  iota = lax.iota(jnp.int32, 16)

  @plsc.parallel_loop(0, BW // 16, unroll=2)
  def _(k):
    ids16 = plsc.load_gather(
        idx_v, [jnp.full((16,), f, jnp.int32), iota + (c * BW + k * 16)])
    basecol = (ids16 & 1) << 6
    rowv = iota + k * 16
    for d in range(DIM):
      t[d, pl.ds(k * 16, 16)] = plsc.load_gather(g, [rowv, basecol + d])


def _make_emb():
  mesh = plsc.VectorSubcoreMesh(core_axis_name="c", subcore_axis_name="s")

  @functools.partial(
      pl.kernel,
      mesh=mesh,
      out_type=jax.ShapeDtypeStruct((FIELDS, DIM, BATCH), jnp.float32),
      scratch_types=[
          pltpu.VMEM((FIELDS, BLK), jnp.int32),
          pltpu.VMEM((FIELDS * BLK,), jnp.int32),
          [pltpu.VMEM((BW, 128), jnp.float32) for _ in range(2)],
          [pltpu.VMEM((DIM, BW), jnp.float32) for _ in range(2)],
          [pltpu.SemaphoreType.DMA for _ in range(2)],
          [pltpu.SemaphoreType.DMA for _ in range(2)],
      ],
      compiler_params=pltpu.CompilerParams(needs_layout_passes=False),
  )
  def body(ids_hbm, jdx_hbm, w2_hbm, out_hbm, idx_v, jdx_v, gb, tb, gsems,
           wsems):
    wid = lax.axis_index("s") * NC + lax.axis_index("c")
    b0 = wid * BLK
    pltpu.sync_copy(ids_hbm.at[:, pl.ds(b0, BLK)], idx_v)
    for f in range(FIELDS):
      pltpu.async_copy(
          jdx_hbm.at[f, pl.ds(b0, BLK)], jdx_v.at[pl.ds(f * BLK, BLK)],
          wsems[0])
    for f in range(FIELDS):
      pltpu.make_async_copy(
          jdx_hbm.at[0, pl.ds(0, BLK)], jdx_v.at[pl.ds(0, BLK)],
          wsems[0]).wait()
    for p in range(2):
      pltpu.async_copy(
          w2_hbm.at[jdx_v.at[pl.ds(p * BW, BW)]], gb[p], gsems[p])

    @pl.loop(0, NTASK, step=2)
    def _(t0):
      f = t0 // 2
      for p in range(2):
        pltpu.make_async_copy(w2_hbm.at[pl.ds(0, BW)], gb[p], gsems[p]).wait()

        @pl.when(t0 + p >= 2)
        def _():
          pltpu.make_async_copy(
              tb[p], out_hbm.at[0, :, pl.ds(0, BW)], wsems[p]).wait()

        _transpose_chunk(idx_v, f, p, gb[p], tb[p])

        @pl.when(f + 1 < FIELDS)
        def _():
          pltpu.async_copy(
              w2_hbm.at[jdx_v.at[pl.ds((f + 1) * BLK + p * BW, BW)]],
              gb[p], gsems[p])

        pltpu.async_copy(
            tb[p], out_hbm.at[f, :, pl.ds(b0 + p * BW, BW)], wsems[p])

    for p in range(2):
      pltpu.make_async_copy(
          tb[p], out_hbm.at[0, :, pl.ds(0, BW)], wsems[p]).wait()

  return body


def kernel(input_ids, weight):
  ids_t = input_ids.T.astype(jnp.int32)
  jdx = lax.shift_right_logical(ids_t, 1)
  w2 = weight.reshape(-1, 128)
  out = _make_emb()(ids_t, jdx, w2)
  return out.transpose(2, 0, 1)


# speedup vs baseline: 1.3186x; 1.0608x over previous
import functools
import jax
import jax.numpy as jnp
from jax import lax
from jax.experimental import pallas as pl
from jax.experimental.pallas import tpu as pltpu
from jax.experimental.pallas import tpu_sc as plsc

BATCH = 16384
FIELDS = 26
DIM = 64
NC = 2
NS = 16
NW = NC * NS
BLK = BATCH // NW
BW = 256
NCHUNK = BLK // BW
NTASK = FIELDS * NCHUNK


def _transpose_chunk(idx_v, f, c, g, t):
  iota = lax.iota(jnp.int32, 16)

  @plsc.parallel_loop(0, BW // 16, unroll=2)
  def _(k):
    ids16 = plsc.load_gather(
        idx_v, [jnp.full((16,), f, jnp.int32), iota + (c * BW + k * 16)])
    basecol = (ids16 & 1) << 6
    rowv = iota + k * 16

    @plsc.parallel_loop(0, DIM, unroll=8)
    def _(d):
      t[d, pl.ds(k * 16, 16)] = plsc.load_gather(g, [rowv, basecol + d])


def _make_emb():
  mesh = plsc.VectorSubcoreMesh(core_axis_name="c", subcore_axis_name="s")

  @functools.partial(
      pl.kernel,
      mesh=mesh,
      out_type=jax.ShapeDtypeStruct((FIELDS, DIM, BATCH), jnp.float32),
      scratch_types=[
          pltpu.VMEM((FIELDS, BLK), jnp.int32),
          pltpu.VMEM((FIELDS * BLK,), jnp.int32),
          [pltpu.VMEM((BW, 128), jnp.float32) for _ in range(2)],
          [pltpu.VMEM((DIM, BW), jnp.float32) for _ in range(2)],
          [pltpu.SemaphoreType.DMA for _ in range(2)],
          [pltpu.SemaphoreType.DMA for _ in range(2)],
      ],
      compiler_params=pltpu.CompilerParams(needs_layout_passes=False),
  )
  def body(ids_hbm, jdx_hbm, w2_hbm, out_hbm, idx_v, jdx_v, gb, tb, gsems,
           wsems):
    wid = lax.axis_index("s") * NC + lax.axis_index("c")
    b0 = wid * BLK
    pltpu.sync_copy(ids_hbm.at[:, pl.ds(b0, BLK)], idx_v)
    for f in range(FIELDS):
      pltpu.async_copy(
          jdx_hbm.at[f, pl.ds(b0, BLK)], jdx_v.at[pl.ds(f * BLK, BLK)],
          wsems[0])
    for f in range(FIELDS):
      pltpu.make_async_copy(
          jdx_hbm.at[0, pl.ds(0, BLK)], jdx_v.at[pl.ds(0, BLK)],
          wsems[0]).wait()
    for p in range(2):
      pltpu.async_copy(
          w2_hbm.at[jdx_v.at[pl.ds(p * BW, BW)]], gb[p], gsems[p])

    @pl.loop(0, NTASK, step=2)
    def _(t0):
      f = t0 // 2
      for p in range(2):
        pltpu.make_async_copy(w2_hbm.at[pl.ds(0, BW)], gb[p], gsems[p]).wait()

        @pl.when(t0 + p >= 2)
        def _():
          pltpu.make_async_copy(
              tb[p], out_hbm.at[0, :, pl.ds(0, BW)], wsems[p]).wait()

        _transpose_chunk(idx_v, f, p, gb[p], tb[p])

        @pl.when(f + 1 < FIELDS)
        def _():
          pltpu.async_copy(
              w2_hbm.at[jdx_v.at[pl.ds((f + 1) * BLK + p * BW, BW)]],
              gb[p], gsems[p])

        pltpu.async_copy(
            tb[p], out_hbm.at[f, :, pl.ds(b0 + p * BW, BW)], wsems[p])

    for p in range(2):
      pltpu.make_async_copy(
          tb[p], out_hbm.at[0, :, pl.ds(0, BW)], wsems[p]).wait()

  return body


def kernel(input_ids, weight):
  ids_t = input_ids.T.astype(jnp.int32)
  jdx = lax.shift_right_logical(ids_t, 1)
  w2 = weight.reshape(-1, 128)
  out = _make_emb()(ids_t, jdx, w2)
  return out.transpose(2, 0, 1)


# speedup vs baseline: 1.3830x; 1.0488x over previous
import functools
import jax
import jax.numpy as jnp
from jax import lax
from jax.experimental import pallas as pl
from jax.experimental.pallas import tpu as pltpu
from jax.experimental.pallas import tpu_sc as plsc

BATCH = 16384
FIELDS = 26
DIM = 64
NC = 2
NS = 16
NW = NC * NS
BLK = BATCH // NW
BW = 128
CPB = BLK // BW
NTASK = FIELDS * CPB
NBUF = 4


def _make_emb():
  mesh = plsc.VectorSubcoreMesh(core_axis_name="c", subcore_axis_name="s")

  @functools.partial(
      pl.kernel,
      mesh=mesh,
      out_type=jax.ShapeDtypeStruct((FIELDS, BATCH, 2 * DIM), jnp.float32),
      scratch_types=[
          pltpu.VMEM((FIELDS * BLK,), jnp.int32),
          [pltpu.VMEM((BW, 2 * DIM), jnp.float32) for _ in range(NBUF)],
          [pltpu.SemaphoreType.DMA for _ in range(NBUF)],
          [pltpu.SemaphoreType.DMA for _ in range(NBUF)],
          pltpu.SemaphoreType.DMA,
      ],
      compiler_params=pltpu.CompilerParams(needs_layout_passes=False),
  )
  def body(ids_hbm, w_hbm, out_hbm, idx_v, gb, gsems, wsems, isem):
    wid = lax.axis_index("s") * NC + lax.axis_index("c")
    b0 = wid * BLK
    for f in range(FIELDS):
      pltpu.async_copy(
          ids_hbm.at[f, pl.ds(b0, BLK)], idx_v.at[pl.ds(f * BLK, BLK)], isem)
    for f in range(FIELDS):
      pltpu.make_async_copy(
          ids_hbm.at[0, pl.ds(0, BLK)], idx_v.at[pl.ds(0, BLK)], isem).wait()

    def idx_slice(t):
      return idx_v.at[pl.ds((t // CPB) * BLK + (t % CPB) * BW, BW)]

    for p in range(NBUF - 1):
      pltpu.async_copy(w_hbm.at[idx_slice(p)], gb[p], gsems[p])

    @pl.loop(0, NTASK, step=NBUF)
    def _(t0):
      for p in range(NBUF):
        t = t0 + p
        bg = (p + NBUF - 1) % NBUF

        @pl.when(jnp.logical_and(t >= 1, t + NBUF - 1 < NTASK))
        def _():
          pltpu.make_async_copy(
              gb[bg], out_hbm.at[0, pl.ds(0, BW), :], wsems[bg]).wait()

        @pl.when(t + NBUF - 1 < NTASK)
        def _():
          g = t + NBUF - 1
          pltpu.async_copy(
              w_hbm.at[idx_v.at[pl.ds((g // CPB) * BLK + (g % CPB) * BW, BW)]],
              gb[bg], gsems[bg])

        pltpu.make_async_copy(
            w_hbm.at[pl.ds(0, BW)], gb[p], gsems[p]).wait()
        f = t // CPB
        pltpu.async_copy(
            gb[p], out_hbm.at[f, pl.ds(b0 + (t % CPB) * BW, BW), :], wsems[p])

    for p in range(NBUF):
      pltpu.make_async_copy(
          gb[p], out_hbm.at[0, pl.ds(0, BW), :], wsems[p]).wait()

  return body


def kernel(input_ids, weight):
  ids_t = input_ids.T.astype(jnp.int32)
  wp = jnp.pad(weight, ((0, 0), (0, DIM)))
  out = _make_emb()(ids_t, wp)
  return out[:, :, :DIM].transpose(1, 0, 2)


# speedup vs baseline: 1.6287x; 1.1777x over previous
import functools
import jax
import jax.numpy as jnp
from jax import lax
from jax.experimental import pallas as pl
from jax.experimental.pallas import tpu as pltpu
from jax.experimental.pallas import tpu_sc as plsc

BATCH = 16384
FIELDS = 26
DIM = 64
NC = 2
NS = 16
NW = NC * NS
BLK = BATCH // NW
BW = 128
CPB = BLK // BW
NTASK = FIELDS * CPB
NBUF = 4


def _make_emb():
  mesh = plsc.VectorSubcoreMesh(core_axis_name="c", subcore_axis_name="s")

  @functools.partial(
      pl.kernel,
      mesh=mesh,
      out_type=jax.ShapeDtypeStruct((FIELDS, BATCH, DIM), jnp.float32),
      scratch_types=[
          pltpu.VMEM((FIELDS * BLK,), jnp.int32),
          [pltpu.VMEM((BW, 2 * DIM), jnp.float32) for _ in range(NBUF)],
          [pltpu.VMEM((BW, DIM), jnp.float32) for _ in range(2)],
          [pltpu.SemaphoreType.DMA for _ in range(NBUF)],
          [pltpu.SemaphoreType.DMA for _ in range(2)],
          pltpu.SemaphoreType.DMA,
      ],
      compiler_params=pltpu.CompilerParams(needs_layout_passes=False),
  )
  def body(ids_hbm, w_hbm, out_hbm, idx_v, gb, cb, gsems, wsems, isem):
    wid = lax.axis_index("s") * NC + lax.axis_index("c")
    b0 = wid * BLK
    for f in range(FIELDS):
      pltpu.async_copy(
          ids_hbm.at[f, pl.ds(b0, BLK)], idx_v.at[pl.ds(f * BLK, BLK)], isem)
    for f in range(FIELDS):
      pltpu.make_async_copy(
          ids_hbm.at[0, pl.ds(0, BLK)], idx_v.at[pl.ds(0, BLK)], isem).wait()

    def idx_slice(t):
      return idx_v.at[pl.ds((t // CPB) * BLK + (t % CPB) * BW, BW)]

    for p in range(NBUF - 1):
      pltpu.async_copy(w_hbm.at[idx_slice(p)], gb[p], gsems[p])

    @pl.loop(0, NTASK, step=NBUF)
    def _(t0):
      for p in range(NBUF):
        t = t0 + p
        q = p % 2

        pltpu.make_async_copy(
            w_hbm.at[pl.ds(0, BW)], gb[p], gsems[p]).wait()

        @pl.when(t + NBUF - 1 < NTASK)
        def _():
          g = t + NBUF - 1
          pltpu.async_copy(
              w_hbm.at[idx_v.at[pl.ds((g // CPB) * BLK + (g % CPB) * BW, BW)]],
              gb[(p + NBUF - 1) % NBUF], gsems[(p + NBUF - 1) % NBUF])

        @pl.when(t >= 2)
        def _():
          pltpu.make_async_copy(
              cb[q], out_hbm.at[0, pl.ds(0, BW), :], wsems[q]).wait()

        @plsc.parallel_loop(0, BW, unroll=4)
        def _(r):
          for m in range(DIM // 16):
            cb[q][r, pl.ds(m * 16, 16)] = gb[p][r, pl.ds(m * 16, 16)]

        f = t // CPB
        pltpu.async_copy(
            cb[q], out_hbm.at[f, pl.ds(b0 + (t % CPB) * BW, BW), :], wsems[q])

    for q in range(2):
      pltpu.make_async_copy(
          cb[q], out_hbm.at[0, pl.ds(0, BW), :], wsems[q]).wait()

  return body


def kernel(input_ids, weight):
  ids_t = input_ids.T.astype(jnp.int32)
  wp = jnp.pad(weight, ((0, 0), (0, DIM)))
  out = _make_emb()(ids_t, wp)
  return out.transpose(1, 0, 2)


# speedup vs baseline: 1.6301x; 1.0009x over previous
import functools
import jax
import jax.numpy as jnp
from jax import lax
from jax.experimental import pallas as pl
from jax.experimental.pallas import tpu as pltpu
from jax.experimental.pallas import tpu_sc as plsc

BATCH = 16384
FIELDS = 26
DIM = 64
NC = 2
NS = 16
NW = NC * NS
BLK = BATCH // NW
BW = 128
CPB = BLK // BW
NTASK = FIELDS * CPB
NBUF = 4


def _make_emb():
  mesh = plsc.VectorSubcoreMesh(core_axis_name="c", subcore_axis_name="s")

  @functools.partial(
      pl.kernel,
      mesh=mesh,
      out_type=jax.ShapeDtypeStruct((FIELDS, BATCH, DIM), jnp.float32),
      scratch_types=[
          pltpu.VMEM((FIELDS * BLK,), jnp.int32),
          [pltpu.VMEM((BW, 2 * DIM), jnp.float32) for _ in range(NBUF)],
          [pltpu.VMEM((BW, DIM), jnp.float32) for _ in range(2)],
          [pltpu.SemaphoreType.DMA for _ in range(NBUF)],
          [pltpu.SemaphoreType.DMA for _ in range(2)],
          pltpu.SemaphoreType.DMA,
      ],
      compiler_params=pltpu.CompilerParams(needs_layout_passes=False),
  )
  def body(ids_hbm, w_hbm, out_hbm, idx_v, gb, cb, gsems, wsems, isem):
    wid = lax.axis_index("s") * NC + lax.axis_index("c")
    b0 = wid * BLK
    for f in range(FIELDS):
      pltpu.async_copy(
          ids_hbm.at[f, pl.ds(b0, BLK)], idx_v.at[pl.ds(f * BLK, BLK)], isem)
    for f in range(FIELDS):
      pltpu.make_async_copy(
          ids_hbm.at[0, pl.ds(0, BLK)], idx_v.at[pl.ds(0, BLK)], isem).wait()

    def idx_slice(t):
      return idx_v.at[pl.ds((t // CPB) * BLK + (t % CPB) * BW, BW)]

    for p in range(NBUF):
      pltpu.async_copy(w_hbm.at[idx_slice(p)], gb[p], gsems[p])

    @pl.loop(0, NTASK, step=NBUF)
    def _(t0):
      for p in range(NBUF):
        t = t0 + p
        q = p % 2

        pltpu.make_async_copy(
            w_hbm.at[pl.ds(0, BW)], gb[p], gsems[p]).wait()

        @pl.when(t >= 2)
        def _():
          pltpu.make_async_copy(
              cb[q], out_hbm.at[0, pl.ds(0, BW), :], wsems[q]).wait()

        @plsc.parallel_loop(0, BW, unroll=4)
        def _(r):
          for m in range(DIM // 16):
            cb[q][r, pl.ds(m * 16, 16)] = gb[p][r, pl.ds(m * 16, 16)]

        @pl.when(t + NBUF < NTASK)
        def _():
          g = t + NBUF
          pltpu.async_copy(
              w_hbm.at[idx_v.at[pl.ds((g // CPB) * BLK + (g % CPB) * BW, BW)]],
              gb[p], gsems[p])

        f = t // CPB
        pltpu.async_copy(
            cb[q], out_hbm.at[f, pl.ds(b0 + (t % CPB) * BW, BW), :], wsems[q])

    for q in range(2):
      pltpu.make_async_copy(
          cb[q], out_hbm.at[0, pl.ds(0, BW), :], wsems[q]).wait()

  return body


def kernel(input_ids, weight):
  ids_t = input_ids.T.astype(jnp.int32)
  wp = jnp.pad(weight, ((0, 0), (0, DIM)))
  out = _make_emb()(ids_t, wp)
  return out.transpose(1, 0, 2)
